# baseline (speedup 1.0000x reference)
import functools

import numpy as np
import jax
import jax.numpy as jnp
from jax import lax
from jax.experimental import pallas as pl
from jax.experimental.pallas import tpu as pltpu
from jax.experimental.pallas import tpu_sc as plsc

D_M = 64
L_SEQ = 200
ENC_BASE = 10000.0
G = 128


def _pos_table_np():
    pos = np.arange(L_SEQ, dtype=np.float32)[:, None]
    i = np.arange(D_M // 2, dtype=np.float32)[None, :]
    denoms = pos / np.power(np.float32(ENC_BASE), 2.0 * i / np.float32(D_M))
    mat = np.zeros((L_SEQ, D_M), dtype=np.float32)
    mat[:, 0::2] = np.sin(denoms)
    mat[:, 1::2] = np.cos(denoms)
    return mat


@functools.lru_cache(maxsize=None)
def _make_sc_kernel(BL, V):
    info = plsc.get_sparse_core_info()
    NC, NS, LANES = info.num_cores, info.num_subcores, info.num_lanes
    NW = NC * NS
    rows_per_w = BL // NW
    n_chunks = rows_per_w // G

    mesh = plsc.VectorSubcoreMesh(core_axis_name="c", subcore_axis_name="s")

    @functools.partial(
        pl.kernel,
        mesh=mesh,
        compiler_params=pltpu.CompilerParams(use_tc_tiling_on_sc=False),
        out_type=jax.ShapeDtypeStruct((BL, D_M), jnp.float32),
        scratch_types=[
            pltpu.VMEM((1, G), jnp.int32),
            pltpu.VMEM((G, D_M), jnp.float32),
            pltpu.VMEM((L_SEQ, D_M), jnp.float32),
            pltpu.SemaphoreType.DMA,
        ],
    )
    def k(x_hbm, pos_hbm, table_hbm, out_hbm, idx_v, rows_v, pos_v, sem):
        wid = lax.axis_index("s") * NC + lax.axis_index("c")
        pltpu.sync_copy(pos_hbm, pos_v)
        wbase = wid * rows_per_w

        def chunk_body(c, carry):
            base = wbase + c * G
            pltpu.sync_copy(x_hbm.at[pl.ds(wid * n_chunks + c, 1)], idx_v)
            pltpu.async_copy(table_hbm.at[idx_v.at[0]], rows_v, sem).wait()

            def add_row(r, l):
                for kk in range(D_M // LANES):
                    s = pl.ds(kk * LANES, LANES)
                    rows_v[r, s] = rows_v[r, s] + pos_v[l, s]
                l = l + 1
                return lax.select(l == L_SEQ, 0, l)

            l0 = lax.rem(base, L_SEQ)
            lax.fori_loop(0, G, add_row, l0)
            pltpu.sync_copy(rows_v, out_hbm.at[pl.ds(base, G)])
            return carry

        lax.fori_loop(0, n_chunks, chunk_body, 0)

    return k


def kernel(x, table):
    B, L = x.shape
    BL = B * L
    x2 = x.reshape(BL // G, G)
    pos = jnp.asarray(_pos_table_np())
    k = _make_sc_kernel(BL, table.shape[0])
    out = k(x2, pos, table)
    return out.reshape(B, L, D_M)


# speedup vs baseline: 1.1867x; 1.1867x over previous
import functools

import numpy as np
import jax
import jax.numpy as jnp
from jax import lax
from jax.experimental import pallas as pl
from jax.experimental.pallas import tpu as pltpu
from jax.experimental.pallas import tpu_sc as plsc

D_M = 64
L_SEQ = 200
ENC_BASE = 10000.0
G = 128
CH = 512
NG = CH // G


def _pos_table_np():
    pos = np.arange(L_SEQ, dtype=np.float32)[:, None]
    i = np.arange(D_M // 2, dtype=np.float32)[None, :]
    denoms = pos / np.power(np.float32(ENC_BASE), 2.0 * i / np.float32(D_M))
    mat = np.zeros((L_SEQ, D_M), dtype=np.float32)
    mat[:, 0::2] = np.sin(denoms)
    mat[:, 1::2] = np.cos(denoms)
    return mat


@functools.lru_cache(maxsize=None)
def _make_sc_kernel(BL, V):
    info = plsc.get_sparse_core_info()
    NC, NS, LANES = info.num_cores, info.num_subcores, info.num_lanes
    NW = NC * NS
    rows_per_w = BL // NW
    n_chunks = rows_per_w // CH
    n_idx_rows = rows_per_w // G

    mesh = plsc.VectorSubcoreMesh(core_axis_name="c", subcore_axis_name="s")

    @functools.partial(
        pl.kernel,
        mesh=mesh,
        compiler_params=pltpu.CompilerParams(use_tc_tiling_on_sc=False),
        out_type=jax.ShapeDtypeStruct((BL, D_M), jnp.float32),
        scratch_types=[
            pltpu.VMEM((n_idx_rows, G), jnp.int32),
            pltpu.VMEM((L_SEQ, D_M), jnp.float32),
            pltpu.VMEM((2, CH, D_M), jnp.float32),
            pltpu.SemaphoreType.DMA,
            pltpu.SemaphoreType.DMA,
            pltpu.SemaphoreType.DMA,
            pltpu.SemaphoreType.DMA,
        ],
    )
    def k(x_hbm, pos_hbm, table_hbm, out_hbm, idx_v, pos_v, rows_v,
          gsem0, gsem1, wsem0, wsem1):
        wid = lax.axis_index("s") * NC + lax.axis_index("c")
        pltpu.sync_copy(pos_hbm, pos_v)
        pltpu.sync_copy(x_hbm.at[pl.ds(wid * n_idx_rows, n_idx_rows)], idx_v)
        wbase = wid * rows_per_w

        def start_gathers(c, b, gsem):
            for j in range(NG):
                pltpu.async_copy(
                    table_hbm.at[idx_v.at[c * NG + j]],
                    rows_v.at[b, pl.ds(j * G, G)],
                    gsem,
                )

        def sub_step(c, b, gsem, wsem, gsem_n, wsem_n):
            pltpu.make_async_copy(
                out_hbm.at[pl.ds(wbase + c * CH, CH)], rows_v.at[b], gsem
            ).wait()

            @pl.when(c >= 1)
            def _():
                pltpu.make_async_copy(
                    rows_v.at[1 - b],
                    out_hbm.at[pl.ds(wbase + (c - 1) * CH, CH)],
                    wsem_n,
                ).wait()

            @pl.when(c + 1 < n_chunks)
            def _():
                start_gathers(c + 1, 1 - b, gsem_n)

            def add_row(r, l):
                for kk in range(D_M // LANES):
                    s = pl.ds(kk * LANES, LANES)
                    rows_v[b, r, s] = rows_v[b, r, s] + pos_v[l, s]
                l = l + 1
                return lax.select(l == L_SEQ, 0, l)

            l0 = lax.rem(c * CH, L_SEQ)
            lax.fori_loop(0, CH, add_row, l0)

            pltpu.async_copy(
                rows_v.at[b], out_hbm.at[pl.ds(wbase + c * CH, CH)], wsem
            )

        start_gathers(0, 0, gsem0)

        def pair_body(i, carry):
            sub_step(2 * i, 0, gsem0, wsem0, gsem1, wsem1)
            sub_step(2 * i + 1, 1, gsem1, wsem1, gsem0, wsem0)
            return carry

        lax.fori_loop(0, n_chunks // 2, pair_body, 0)

        pltpu.make_async_copy(
            rows_v.at[1],
            out_hbm.at[pl.ds(wbase + (n_chunks - 1) * CH, CH)],
            wsem1,
        ).wait()

    return k


def kernel(x, table):
    B, L = x.shape
    BL = B * L
    x2 = x.reshape(BL // G, G)
    pos = jnp.asarray(_pos_table_np())
    k = _make_sc_kernel(BL, table.shape[0])
    out = k(x2, pos, table)
    return out.reshape(B, L, D_M)
